# baseline (speedup 1.0000x reference)
import functools

import jax
import jax.numpy as jnp
from jax import lax
from jax.experimental import pallas as pl
from jax.experimental.pallas import tpu as pltpu
from jax.experimental.pallas import tpu_sc as plsc

NC = 2
NS = 16
NW = NC * NS

F32 = jnp.float32


def _relu_proj_body(x_ref, w_ref, b_ref, o_ref):
    o_ref[...] = jax.nn.relu(
        jnp.dot(x_ref[...], w_ref[...], preferred_element_type=F32) + b_ref[...])


def _relu_proj(x, w, b, block_rows):
    n, d_in = x.shape
    d_out = w.shape[1]
    grid = n // block_rows
    return pl.pallas_call(
        _relu_proj_body,
        grid=(grid,),
        in_specs=[
            pl.BlockSpec((block_rows, d_in), lambda i: (i, 0)),
            pl.BlockSpec((d_in, d_out), lambda i: (0, 0)),
            pl.BlockSpec((1, d_out), lambda i: (0, 0)),
        ],
        out_specs=pl.BlockSpec((block_rows, d_out), lambda i: (i, 0)),
        out_shape=jax.ShapeDtypeStruct((n, d_out), F32),
    )(x, w, b.reshape(1, d_out))


def _ln(y, g, b):
    m = jnp.mean(y, axis=-1, keepdims=True)
    v = jnp.mean((y - m) ** 2, axis=-1, keepdims=True)
    return (y - m) * lax.rsqrt(v + 1e-5) * g + b


def _edge_layer_body(hs_ref, hd_ref, ea_ref, eh_ref, w1s_ref, w1d_ref, w1e_ref,
                     b1_ref, w2_ref, b2_ref, g_ref, bb_ref, o_ref):
    dt = w1s_ref.dtype
    h = jnp.dot(hs_ref[...].astype(dt), w1s_ref[...], preferred_element_type=F32)
    h = h + jnp.dot(hd_ref[...].astype(dt), w1d_ref[...],
                    preferred_element_type=F32)
    h = h + jnp.dot(ea_ref[...], w1e_ref[...], preferred_element_type=F32)
    h = jax.nn.relu(h + b1_ref[...])
    msg = jnp.dot(h.astype(w2_ref.dtype), w2_ref[...],
                  preferred_element_type=F32) + b2_ref[...]
    y = eh_ref[...] + msg
    o_ref[...] = _ln(y, g_ref[...], bb_ref[...])


def _edge_layer(hs, hd, ea, eh, w1s, w1d, w1e, b1, w2, b2, g, bb, block_rows):
    e, hdim = eh.shape
    hp = hs.shape[1]
    de = ea.shape[1]
    h2 = w2.shape[0]
    grid = e // block_rows
    bf16 = jnp.bfloat16
    return pl.pallas_call(
        _edge_layer_body,
        grid=(grid,),
        in_specs=[
            pl.BlockSpec((block_rows, hp), lambda i: (i, 0)),
            pl.BlockSpec((block_rows, hp), lambda i: (i, 0)),
            pl.BlockSpec((block_rows, de), lambda i: (i, 0)),
            pl.BlockSpec((block_rows, hdim), lambda i: (i, 0)),
            pl.BlockSpec((hp, h2), lambda i: (0, 0)),
            pl.BlockSpec((hp, h2), lambda i: (0, 0)),
            pl.BlockSpec((de, h2), lambda i: (0, 0)),
            pl.BlockSpec((1, h2), lambda i: (0, 0)),
            pl.BlockSpec((h2, hdim), lambda i: (0, 0)),
            pl.BlockSpec((1, hdim), lambda i: (0, 0)),
            pl.BlockSpec((1, hdim), lambda i: (0, 0)),
            pl.BlockSpec((1, hdim), lambda i: (0, 0)),
        ],
        out_specs=pl.BlockSpec((block_rows, hdim), lambda i: (i, 0)),
        out_shape=jax.ShapeDtypeStruct((e, hdim), F32),
    )(hs, hd, ea, eh,
      w1s.astype(bf16), w1d.astype(bf16),
      w1e, b1.reshape(1, h2), w2,
      b2.reshape(1, hdim), g.reshape(1, hdim), bb.reshape(1, hdim))


def _node_layer_body(nh_ref, agg_ref, w1n_ref, w1a_ref, b1_ref, w2_ref, b2_ref,
                     g_ref, bb_ref, o_ref):
    agg = agg_ref[0] + agg_ref[1]
    nh = nh_ref[...]
    h = jnp.dot(nh, w1n_ref[...], preferred_element_type=F32)
    h = h + jnp.dot(agg, w1a_ref[...], preferred_element_type=F32)
    h = jax.nn.relu(h + b1_ref[...])
    upd = jnp.dot(h, w2_ref[...], preferred_element_type=F32) + b2_ref[...]
    o_ref[...] = _ln(nh + upd, g_ref[...], bb_ref[...])


def _node_layer(nh, aggp, w1n, w1a, b1, w2, b2, g, bb, block_rows):
    n, hdim = nh.shape
    h2 = w2.shape[0]
    grid = n // block_rows
    return pl.pallas_call(
        _node_layer_body,
        grid=(grid,),
        in_specs=[
            pl.BlockSpec((block_rows, hdim), lambda i: (i, 0)),
            pl.BlockSpec((2, block_rows, hdim), lambda i: (0, i, 0)),
            pl.BlockSpec((hdim, h2), lambda i: (0, 0)),
            pl.BlockSpec((hdim, h2), lambda i: (0, 0)),
            pl.BlockSpec((1, h2), lambda i: (0, 0)),
            pl.BlockSpec((h2, hdim), lambda i: (0, 0)),
            pl.BlockSpec((1, hdim), lambda i: (0, 0)),
            pl.BlockSpec((1, hdim), lambda i: (0, 0)),
            pl.BlockSpec((1, hdim), lambda i: (0, 0)),
        ],
        out_specs=pl.BlockSpec((block_rows, hdim), lambda i: (i, 0)),
        out_shape=jax.ShapeDtypeStruct((n, hdim), F32),
    )(nh, aggp, w1n, w1a, b1.reshape(1, h2), w2, b2.reshape(1, hdim),
      g.reshape(1, hdim), bb.reshape(1, hdim))


def _head_body(lat_ref, sidx_ref, emb0_ref, emb1_ref, h0_ref, irr_ref, rea_ref,
               h0w1_ref, h0b1_ref, h0w2_ref, h0b2_ref,
               t1lat_ref, t1h0_ref, tirr_ref, trea_ref, tb1_ref, tw2_ref,
               tb2_ref, s1p_ref, s1h_ref, sb1_ref, sw2_ref, sb2_ref,
               tok_ref, size_ref):
    sf = sidx_ref[...]
    side = sf * emb1_ref[...] + (1.0 - sf) * emb0_ref[...]
    lath = lat_ref[...] + side
    irr = irr_ref[...]
    rea = rea_ref[...]

    h0 = h0_ref[...]
    h0h = jax.nn.relu(h0 * h0w1_ref[...] + h0b1_ref[...])
    h0_emb = jnp.dot(h0h, h0w2_ref[...], preferred_element_type=F32) + h0b2_ref[...]

    base = jnp.dot(lath, t1lat_ref[...], preferred_element_type=F32)
    h0c = jnp.dot(h0_emb, t1h0_ref[...], preferred_element_type=F32)
    w_irr = tirr_ref[...]
    w_rea = trea_ref[...]
    tw2 = tw2_ref[...]
    nb = irr.shape[0]
    for b in range(nb):
        hid = jax.nn.relu(base + h0c[b] + irr[b][:, None] * w_irr
                          + rea[b][:, None] * w_rea + tb1_ref[...])
        tok_ref[b, :] = jnp.sum(hid * tw2, axis=1) + tb2_ref[0, 0]

    avail = rea * (1.0 - irr)
    denom = jnp.maximum(jnp.sum(avail, axis=1, keepdims=True), 1.0)
    pooled = jnp.dot(avail, lath, preferred_element_type=F32) / denom
    sh = jax.nn.relu(jnp.dot(pooled, s1p_ref[...], preferred_element_type=F32)
                     + jnp.dot(h0_emb, s1h_ref[...], preferred_element_type=F32)
                     + sb1_ref[...])
    size_ref[...] = jnp.dot(sh, sw2_ref[...], preferred_element_type=F32) + sb2_ref[...]


def _head(lat, side_idx, h0, irr_f, rea_f, p):
    l, hdim = lat.shape
    nb = h0.shape[0]
    tok_w1 = p['tok_W1']
    emb = p['side_emb']
    return pl.pallas_call(
        _head_body,
        out_shape=(jax.ShapeDtypeStruct((nb, l), F32),
                   jax.ShapeDtypeStruct((nb, 3), F32)),
    )(lat, side_idx.astype(F32).reshape(l, 1), emb[0].reshape(1, hdim),
      emb[1].reshape(1, hdim), h0.reshape(nb, 1), irr_f, rea_f,
      p['h0_W1'].reshape(1, hdim), p['h0_b1'].reshape(1, hdim), p['h0_W2'],
      p['h0_b2'].reshape(1, hdim),
      tok_w1[:hdim], tok_w1[hdim:2 * hdim], tok_w1[2 * hdim].reshape(1, hdim),
      tok_w1[2 * hdim + 1].reshape(1, hdim), p['tok_b1'].reshape(1, hdim),
      p['tok_W2'].reshape(1, hdim), p['tok_b2'].reshape(1, 1),
      p['size_W1'][:hdim], p['size_W1'][hdim:], p['size_b1'].reshape(1, hdim),
      p['size_W2'], p['size_b2'].reshape(1, 3))


def _gather2(node_h, src, dst, chunk=80, nbuf=5):
    e = src.shape[0]
    hdim = node_h.shape[1]
    dt = node_h.dtype
    per_w = e // NW
    n_chunks = per_w // chunk
    n_groups = n_chunks // nbuf
    assert n_chunks % nbuf == 0
    mesh = plsc.VectorSubcoreMesh(core_axis_name="c", subcore_axis_name="s")

    @functools.partial(
        pl.kernel, mesh=mesh,
        out_type=(jax.ShapeDtypeStruct((e, hdim), dt),
                  jax.ShapeDtypeStruct((e, hdim), dt)),
        scratch_types=[
            pltpu.VMEM((per_w,), jnp.int32),
            pltpu.VMEM((per_w,), jnp.int32),
            pltpu.VMEM((nbuf, chunk, hdim), dt),
            pltpu.VMEM((nbuf, chunk, hdim), dt),
            pltpu.SemaphoreType.DMA,
            pltpu.SemaphoreType.DMA,
        ],
    )
    def k(node_hbm, src_hbm, dst_hbm, hs_hbm, hd_hbm,
          sidx, didx, srows, drows, gsem, wsem):
        wid = lax.axis_index("s") * NC + lax.axis_index("c")
        base = wid * per_w
        pltpu.sync_copy(src_hbm.at[pl.ds(base, per_w)], sidx)
        pltpu.sync_copy(dst_hbm.at[pl.ds(base, per_w)], didx)

        def group(g, carry):
            g0 = g * nbuf * chunk
            descs = []
            for b in range(nbuf):
                lo = g0 + b * chunk
                descs.append((
                    pltpu.async_copy(node_hbm.at[sidx.at[pl.ds(lo, chunk)]],
                                     srows.at[b], gsem),
                    pltpu.async_copy(node_hbm.at[didx.at[pl.ds(lo, chunk)]],
                                     drows.at[b], gsem)))
            wdescs = []
            for b in range(nbuf):
                a1, a2 = descs[b]
                a1.wait()
                a2.wait()
                off = base + g0 + b * chunk
                wdescs.append((
                    pltpu.async_copy(srows.at[b], hs_hbm.at[pl.ds(off, chunk)],
                                     wsem),
                    pltpu.async_copy(drows.at[b], hd_hbm.at[pl.ds(off, chunk)],
                                     wsem)))
            for w1, w2 in wdescs:
                w1.wait()
                w2.wait()
            return carry

        lax.fori_loop(0, n_groups, group, 0)

    return k(node_h, src, dst)


def _scatter_add(edge_h, dst, n_pad, zeros_hbm, chunk=40, nbuf=5):
    e, hdim = edge_h.shape
    per_w = e // NW
    n_chunks = per_w // chunk
    rows_per_tile = n_pad // NS
    mesh = plsc.VectorSubcoreMesh(core_axis_name="c", subcore_axis_name="s")

    @functools.partial(
        pl.kernel, mesh=mesh,
        out_type=jax.ShapeDtypeStruct((NC, n_pad, hdim), F32),
        scratch_types=[
            pltpu.VMEM((nbuf, chunk), jnp.int32),
            pltpu.VMEM((nbuf, chunk, hdim), F32),
            pltpu.VMEM_SHARED((n_pad, hdim), F32),
            pltpu.SemaphoreType.DMA,
        ],
    )
    def k(eh_hbm, dst_hbm, zero_hbm, out_hbm, idx_v, rows_v, acc_sh, sem):
        c = lax.axis_index("c")
        s = lax.axis_index("s")
        pltpu.sync_copy(zero_hbm.at[pl.ds(s * rows_per_tile, rows_per_tile)],
                        acc_sh.at[pl.ds(s * rows_per_tile, rows_per_tile)])
        plsc.subcore_barrier()

        base = (c * NS + s) * per_w

        def group(g, carry):
            descs = []
            for b in range(nbuf):
                off = base + (g * nbuf + b) * chunk
                descs.append((
                    pltpu.async_copy(dst_hbm.at[pl.ds(off, chunk)],
                                     idx_v.at[b], sem),
                    pltpu.async_copy(eh_hbm.at[pl.ds(off, chunk)],
                                     rows_v.at[b], sem)))
            for b in range(nbuf):
                a1, a2 = descs[b]
                a1.wait()
                a2.wait()
                pltpu.sync_copy(rows_v.at[b], acc_sh.at[idx_v.at[b]], add=True)
            return carry

        lax.fori_loop(0, n_chunks // nbuf, group, 0)
        plsc.subcore_barrier()
        pltpu.sync_copy(acc_sh.at[pl.ds(s * rows_per_tile, rows_per_tile)],
                        out_hbm.at[c, pl.ds(s * rows_per_tile, rows_per_tile)])

    return k(edge_h, dst, zeros_hbm)


def _gather1(node_h, idx):
    l = idx.shape[0]
    hdim = node_h.shape[1]
    per_w = l // NW
    mesh = plsc.VectorSubcoreMesh(core_axis_name="c", subcore_axis_name="s")

    @functools.partial(
        pl.kernel, mesh=mesh,
        out_type=jax.ShapeDtypeStruct((l, hdim), F32),
        scratch_types=[
            pltpu.VMEM((per_w,), jnp.int32),
            pltpu.VMEM((per_w, hdim), F32),
            pltpu.SemaphoreType.DMA,
        ],
    )
    def k(node_hbm, idx_hbm, out_hbm, idx_v, rows_v, sem):
        wid = lax.axis_index("s") * NC + lax.axis_index("c")
        base = wid * per_w
        pltpu.sync_copy(idx_hbm.at[pl.ds(base, per_w)], idx_v)
        pltpu.async_copy(node_hbm.at[idx_v], rows_v, sem).wait()
        pltpu.sync_copy(rows_v, out_hbm.at[pl.ds(base, per_w)])

    return k(node_h, idx)


def kernel(node_x, edge_index, edge_attr, lateral_to_node_idx, side_idx, H0,
           irrigated, reachable, params):
    p = params
    n, _ = node_x.shape
    e = edge_attr.shape[0]
    hdim = p['node_proj_W'].shape[1]
    nl = p['eW1'].shape[0]

    node_h = _relu_proj(node_x, p['node_proj_W'], p['node_proj_b'], 1000)
    edge_h = _relu_proj(edge_attr, p['edge_proj_W'], p['edge_proj_b'], 4000)

    src = edge_index[0]
    dst = edge_index[1]
    n_pad = ((n + 8 * NS - 1) // (8 * NS)) * (8 * NS)
    zeros_hbm = jnp.zeros((n_pad, hdim), F32)

    bf16 = jnp.bfloat16
    for i in range(nl):
        hs, hd = _gather2(node_h, src, dst)
        ew1 = p['eW1'][i]
        edge_h = _edge_layer(
            hs, hd, edge_attr, edge_h,
            ew1[:hdim], ew1[hdim:2 * hdim],
            ew1[2 * hdim:],
            p['eb1'][i], p['eW2'][i].astype(bf16), p['eb2'][i],
            p['ln_eg'][i], p['ln_eb'][i], 4000)
        aggp = _scatter_add(edge_h, dst, n_pad, zeros_hbm)
        nw1 = p['nW1'][i]
        node_h = _node_layer(
            node_h, aggp, nw1[:hdim], nw1[hdim:],
            p['nb1'][i], p['nW2'][i], p['nb2'][i],
            p['ln_ng'][i], p['ln_nb'][i], 1000)

    lat = _gather1(node_h, lateral_to_node_idx)
    token_logits, size_logits = _head(
        lat, side_idx, H0, irrigated.astype(F32), reachable.astype(F32), p)
    return (token_logits, size_logits)


# speedup vs baseline: 1.0443x; 1.0443x over previous
import functools

import jax
import jax.numpy as jnp
from jax import lax
from jax.experimental import pallas as pl
from jax.experimental.pallas import tpu as pltpu
from jax.experimental.pallas import tpu_sc as plsc

NC = 2
NS = 16
NW = NC * NS

F32 = jnp.float32


def _relu_proj_body(x_ref, w_ref, b_ref, o_ref):
    o_ref[...] = jax.nn.relu(
        jnp.dot(x_ref[...], w_ref[...], preferred_element_type=F32) + b_ref[...])


def _relu_proj(x, w, b, block_rows):
    n, d_in = x.shape
    d_out = w.shape[1]
    grid = n // block_rows
    return pl.pallas_call(
        _relu_proj_body,
        grid=(grid,),
        in_specs=[
            pl.BlockSpec((block_rows, d_in), lambda i: (i, 0)),
            pl.BlockSpec((d_in, d_out), lambda i: (0, 0)),
            pl.BlockSpec((1, d_out), lambda i: (0, 0)),
        ],
        out_specs=pl.BlockSpec((block_rows, d_out), lambda i: (i, 0)),
        out_shape=jax.ShapeDtypeStruct((n, d_out), F32),
    )(x, w, b.reshape(1, d_out))


def _ln(y, g, b):
    m = jnp.mean(y, axis=-1, keepdims=True)
    v = jnp.mean((y - m) ** 2, axis=-1, keepdims=True)
    return (y - m) * lax.rsqrt(v + 1e-5) * g + b


def _edge_layer_body(hs_ref, hd_ref, ea_ref, eh_ref, w1s_ref, w1d_ref, w1e_ref,
                     b1_ref, w2_ref, b2_ref, g_ref, bb_ref, o_ref):
    dt = w1s_ref.dtype
    h = jnp.dot(hs_ref[...].astype(dt), w1s_ref[...], preferred_element_type=F32)
    h = h + jnp.dot(hd_ref[...].astype(dt), w1d_ref[...],
                    preferred_element_type=F32)
    h = h + jnp.dot(ea_ref[...], w1e_ref[...], preferred_element_type=F32)
    h = jax.nn.relu(h + b1_ref[...])
    msg = jnp.dot(h.astype(w2_ref.dtype), w2_ref[...],
                  preferred_element_type=F32) + b2_ref[...]
    y = eh_ref[...] + msg
    o_ref[...] = _ln(y, g_ref[...], bb_ref[...])


def _edge_layer(hs, hd, ea, eh, w1s, w1d, w1e, b1, w2, b2, g, bb, block_rows):
    e, hdim = eh.shape
    hp = hs.shape[1]
    de = ea.shape[1]
    h2 = w2.shape[0]
    grid = e // block_rows
    bf16 = jnp.bfloat16
    return pl.pallas_call(
        _edge_layer_body,
        grid=(grid,),
        in_specs=[
            pl.BlockSpec((block_rows, hp), lambda i: (i, 0)),
            pl.BlockSpec((block_rows, hp), lambda i: (i, 0)),
            pl.BlockSpec((block_rows, de), lambda i: (i, 0)),
            pl.BlockSpec((block_rows, hdim), lambda i: (i, 0)),
            pl.BlockSpec((hp, h2), lambda i: (0, 0)),
            pl.BlockSpec((hp, h2), lambda i: (0, 0)),
            pl.BlockSpec((de, h2), lambda i: (0, 0)),
            pl.BlockSpec((1, h2), lambda i: (0, 0)),
            pl.BlockSpec((h2, hdim), lambda i: (0, 0)),
            pl.BlockSpec((1, hdim), lambda i: (0, 0)),
            pl.BlockSpec((1, hdim), lambda i: (0, 0)),
            pl.BlockSpec((1, hdim), lambda i: (0, 0)),
        ],
        out_specs=pl.BlockSpec((block_rows, hdim), lambda i: (i, 0)),
        out_shape=jax.ShapeDtypeStruct((e, hdim), F32),
    )(hs, hd, ea, eh,
      w1s.astype(bf16), w1d.astype(bf16),
      w1e, b1.reshape(1, h2), w2,
      b2.reshape(1, hdim), g.reshape(1, hdim), bb.reshape(1, hdim))


def _node_layer_body(nh_ref, agg_ref, aggb_ref, w1n_ref, w1a_ref, b1_ref,
                     w2_ref, b2_ref, g_ref, bb_ref, o_ref):
    agg = agg_ref[0] + agg_ref[1] + aggb_ref[0] + aggb_ref[1]
    nh = nh_ref[...]
    h = jnp.dot(nh, w1n_ref[...], preferred_element_type=F32)
    h = h + jnp.dot(agg, w1a_ref[...], preferred_element_type=F32)
    h = jax.nn.relu(h + b1_ref[...])
    upd = jnp.dot(h, w2_ref[...], preferred_element_type=F32) + b2_ref[...]
    o_ref[...] = _ln(nh + upd, g_ref[...], bb_ref[...])


def _node_layer(nh, aggp, aggp2, w1n, w1a, b1, w2, b2, g, bb, block_rows):
    n, hdim = nh.shape
    h2 = w2.shape[0]
    grid = n // block_rows
    return pl.pallas_call(
        _node_layer_body,
        grid=(grid,),
        in_specs=[
            pl.BlockSpec((block_rows, hdim), lambda i: (i, 0)),
            pl.BlockSpec((2, block_rows, hdim), lambda i: (0, i, 0)),
            pl.BlockSpec((2, block_rows, hdim), lambda i: (0, i, 0)),
            pl.BlockSpec((hdim, h2), lambda i: (0, 0)),
            pl.BlockSpec((hdim, h2), lambda i: (0, 0)),
            pl.BlockSpec((1, h2), lambda i: (0, 0)),
            pl.BlockSpec((h2, hdim), lambda i: (0, 0)),
            pl.BlockSpec((1, hdim), lambda i: (0, 0)),
            pl.BlockSpec((1, hdim), lambda i: (0, 0)),
            pl.BlockSpec((1, hdim), lambda i: (0, 0)),
        ],
        out_specs=pl.BlockSpec((block_rows, hdim), lambda i: (i, 0)),
        out_shape=jax.ShapeDtypeStruct((n, hdim), F32),
    )(nh, aggp, aggp2, w1n, w1a, b1.reshape(1, h2), w2, b2.reshape(1, hdim),
      g.reshape(1, hdim), bb.reshape(1, hdim))


def _head_body(lat_ref, sidx_ref, emb0_ref, emb1_ref, h0_ref, irr_ref, rea_ref,
               h0w1_ref, h0b1_ref, h0w2_ref, h0b2_ref,
               t1lat_ref, t1h0_ref, tirr_ref, trea_ref, tb1_ref, tw2_ref,
               tb2_ref, s1p_ref, s1h_ref, sb1_ref, sw2_ref, sb2_ref,
               tok_ref, size_ref):
    sf = sidx_ref[...]
    side = sf * emb1_ref[...] + (1.0 - sf) * emb0_ref[...]
    lath = lat_ref[...] + side
    irr = irr_ref[...]
    rea = rea_ref[...]

    h0 = h0_ref[...]
    h0h = jax.nn.relu(h0 * h0w1_ref[...] + h0b1_ref[...])
    h0_emb = jnp.dot(h0h, h0w2_ref[...], preferred_element_type=F32) + h0b2_ref[...]

    base = jnp.dot(lath, t1lat_ref[...], preferred_element_type=F32)
    h0c = jnp.dot(h0_emb, t1h0_ref[...], preferred_element_type=F32)
    w_irr = tirr_ref[...]
    w_rea = trea_ref[...]
    tw2 = tw2_ref[...]
    nb = irr.shape[0]
    for b in range(nb):
        hid = jax.nn.relu(base + h0c[b] + irr[b][:, None] * w_irr
                          + rea[b][:, None] * w_rea + tb1_ref[...])
        tok_ref[b, :] = jnp.sum(hid * tw2, axis=1) + tb2_ref[0, 0]

    avail = rea * (1.0 - irr)
    denom = jnp.maximum(jnp.sum(avail, axis=1, keepdims=True), 1.0)
    pooled = jnp.dot(avail, lath, preferred_element_type=F32) / denom
    sh = jax.nn.relu(jnp.dot(pooled, s1p_ref[...], preferred_element_type=F32)
                     + jnp.dot(h0_emb, s1h_ref[...], preferred_element_type=F32)
                     + sb1_ref[...])
    size_ref[...] = jnp.dot(sh, sw2_ref[...], preferred_element_type=F32) + sb2_ref[...]


def _head(lat, side_idx, h0, irr_f, rea_f, p):
    l, hdim = lat.shape
    nb = h0.shape[0]
    tok_w1 = p['tok_W1']
    emb = p['side_emb']
    return pl.pallas_call(
        _head_body,
        out_shape=(jax.ShapeDtypeStruct((nb, l), F32),
                   jax.ShapeDtypeStruct((nb, 3), F32)),
    )(lat, side_idx.astype(F32).reshape(l, 1), emb[0].reshape(1, hdim),
      emb[1].reshape(1, hdim), h0.reshape(nb, 1), irr_f, rea_f,
      p['h0_W1'].reshape(1, hdim), p['h0_b1'].reshape(1, hdim), p['h0_W2'],
      p['h0_b2'].reshape(1, hdim),
      tok_w1[:hdim], tok_w1[hdim:2 * hdim], tok_w1[2 * hdim].reshape(1, hdim),
      tok_w1[2 * hdim + 1].reshape(1, hdim), p['tok_b1'].reshape(1, hdim),
      p['tok_W2'].reshape(1, hdim), p['tok_b2'].reshape(1, 1),
      p['size_W1'][:hdim], p['size_W1'][hdim:], p['size_b1'].reshape(1, hdim),
      p['size_W2'], p['size_b2'].reshape(1, 3))


def _gather2(node_h, src, dst, chunk=80, nbuf=5):
    e = src.shape[0]
    hdim = node_h.shape[1]
    dt = node_h.dtype
    per_w = e // NW
    n_chunks = per_w // chunk
    n_groups = n_chunks // nbuf
    assert n_chunks % nbuf == 0
    mesh = plsc.VectorSubcoreMesh(core_axis_name="c", subcore_axis_name="s")

    @functools.partial(
        pl.kernel, mesh=mesh,
        out_type=(jax.ShapeDtypeStruct((e, hdim), dt),
                  jax.ShapeDtypeStruct((e, hdim), dt)),
        scratch_types=[
            pltpu.VMEM((per_w,), jnp.int32),
            pltpu.VMEM((per_w,), jnp.int32),
            pltpu.VMEM((nbuf, chunk, hdim), dt),
            pltpu.VMEM((nbuf, chunk, hdim), dt),
            pltpu.SemaphoreType.DMA,
            pltpu.SemaphoreType.DMA,
        ],
    )
    def k(node_hbm, src_hbm, dst_hbm, hs_hbm, hd_hbm,
          sidx, didx, srows, drows, gsem, wsem):
        wid = lax.axis_index("s") * NC + lax.axis_index("c")
        base = wid * per_w
        pltpu.sync_copy(src_hbm.at[pl.ds(base, per_w)], sidx)
        pltpu.sync_copy(dst_hbm.at[pl.ds(base, per_w)], didx)

        def group(g, carry):
            g0 = g * nbuf * chunk
            descs = []
            for b in range(nbuf):
                lo = g0 + b * chunk
                descs.append((
                    pltpu.async_copy(node_hbm.at[sidx.at[pl.ds(lo, chunk)]],
                                     srows.at[b], gsem),
                    pltpu.async_copy(node_hbm.at[didx.at[pl.ds(lo, chunk)]],
                                     drows.at[b], gsem)))
            wdescs = []
            for b in range(nbuf):
                a1, a2 = descs[b]
                a1.wait()
                a2.wait()
                off = base + g0 + b * chunk
                wdescs.append((
                    pltpu.async_copy(srows.at[b], hs_hbm.at[pl.ds(off, chunk)],
                                     wsem),
                    pltpu.async_copy(drows.at[b], hd_hbm.at[pl.ds(off, chunk)],
                                     wsem)))
            for w1, w2 in wdescs:
                w1.wait()
                w2.wait()
            return carry

        lax.fori_loop(0, n_groups, group, 0)

    return k(node_h, src, dst)


def _scatter_add(edge_h, dst, n_pad, zeros_hbm, chunk=40, nbuf=5):
    e, hdim = edge_h.shape
    per_w = e // NW
    n_chunks = per_w // chunk
    rows_per_tile = n_pad // NS
    mesh = plsc.VectorSubcoreMesh(core_axis_name="c", subcore_axis_name="s")

    @functools.partial(
        pl.kernel, mesh=mesh,
        out_type=jax.ShapeDtypeStruct((NC, n_pad, hdim), F32),
        scratch_types=[
            pltpu.VMEM((nbuf, chunk), jnp.int32),
            pltpu.VMEM((nbuf, chunk, hdim), F32),
            pltpu.VMEM_SHARED((n_pad, hdim), F32),
            pltpu.SemaphoreType.DMA,
        ],
    )
    def k(eh_hbm, dst_hbm, zero_hbm, out_hbm, idx_v, rows_v, acc_sh, sem):
        c = lax.axis_index("c")
        s = lax.axis_index("s")
        pltpu.sync_copy(zero_hbm.at[pl.ds(s * rows_per_tile, rows_per_tile)],
                        acc_sh.at[pl.ds(s * rows_per_tile, rows_per_tile)])
        plsc.subcore_barrier()

        base = (c * NS + s) * per_w

        def group(g, carry):
            descs = []
            for b in range(nbuf):
                off = base + (g * nbuf + b) * chunk
                descs.append((
                    pltpu.async_copy(dst_hbm.at[pl.ds(off, chunk)],
                                     idx_v.at[b], sem),
                    pltpu.async_copy(eh_hbm.at[pl.ds(off, chunk)],
                                     rows_v.at[b], sem)))
            for b in range(nbuf):
                a1, a2 = descs[b]
                a1.wait()
                a2.wait()
                pltpu.sync_copy(rows_v.at[b], acc_sh.at[idx_v.at[b]], add=True)
            return carry

        lax.fori_loop(0, n_chunks // nbuf, group, 0)
        plsc.subcore_barrier()
        pltpu.sync_copy(acc_sh.at[pl.ds(s * rows_per_tile, rows_per_tile)],
                        out_hbm.at[c, pl.ds(s * rows_per_tile, rows_per_tile)])

    return k(edge_h, dst, zeros_hbm)


def _gather1(node_h, idx):
    l = idx.shape[0]
    hdim = node_h.shape[1]
    per_w = l // NW
    mesh = plsc.VectorSubcoreMesh(core_axis_name="c", subcore_axis_name="s")

    @functools.partial(
        pl.kernel, mesh=mesh,
        out_type=jax.ShapeDtypeStruct((l, hdim), F32),
        scratch_types=[
            pltpu.VMEM((per_w,), jnp.int32),
            pltpu.VMEM((per_w, hdim), F32),
            pltpu.SemaphoreType.DMA,
        ],
    )
    def k(node_hbm, idx_hbm, out_hbm, idx_v, rows_v, sem):
        wid = lax.axis_index("s") * NC + lax.axis_index("c")
        base = wid * per_w
        pltpu.sync_copy(idx_hbm.at[pl.ds(base, per_w)], idx_v)
        pltpu.async_copy(node_hbm.at[idx_v], rows_v, sem).wait()
        pltpu.sync_copy(rows_v, out_hbm.at[pl.ds(base, per_w)])

    return k(node_h, idx)


def kernel(node_x, edge_index, edge_attr, lateral_to_node_idx, side_idx, H0,
           irrigated, reachable, params):
    p = params
    n, _ = node_x.shape
    e = edge_attr.shape[0]
    hdim = p['node_proj_W'].shape[1]
    nl = p['eW1'].shape[0]

    node_h = _relu_proj(node_x, p['node_proj_W'], p['node_proj_b'], 1000)

    eh_ = e // 2
    ea_halves = (edge_attr[:eh_], edge_attr[eh_:])
    src_halves = (edge_index[0, :eh_], edge_index[0, eh_:])
    dst_halves = (edge_index[1, :eh_], edge_index[1, eh_:])
    edge_h = [
        _relu_proj(a, p['edge_proj_W'], p['edge_proj_b'], 4000)
        for a in ea_halves
    ]
    n_pad = ((n + 8 * NS - 1) // (8 * NS)) * (8 * NS)
    zeros_hbm = jnp.zeros((n_pad, hdim), F32)

    bf16 = jnp.bfloat16
    for i in range(nl):
        ew1 = p['eW1'][i]
        aggs = []
        for h in range(2):
            hs, hd = _gather2(node_h, src_halves[h], dst_halves[h], chunk=40)
            edge_h[h] = _edge_layer(
                hs, hd, ea_halves[h], edge_h[h],
                ew1[:hdim], ew1[hdim:2 * hdim],
                ew1[2 * hdim:],
                p['eb1'][i], p['eW2'][i].astype(bf16), p['eb2'][i],
                p['ln_eg'][i], p['ln_eb'][i], 4000)
            aggs.append(_scatter_add(edge_h[h], dst_halves[h], n_pad,
                                     zeros_hbm))
        nw1 = p['nW1'][i]
        node_h = _node_layer(
            node_h, aggs[0], aggs[1], nw1[:hdim], nw1[hdim:],
            p['nb1'][i], p['nW2'][i], p['nb2'][i],
            p['ln_ng'][i], p['ln_nb'][i], 1000)

    lat = _gather1(node_h, lateral_to_node_idx)
    token_logits, size_logits = _head(
        lat, side_idx, H0, irrigated.astype(F32), reachable.astype(F32), p)
    return (token_logits, size_logits)


# speedup vs baseline: 1.0621x; 1.0170x over previous
import functools

import jax
import jax.numpy as jnp
from jax import lax
from jax.experimental import pallas as pl
from jax.experimental.pallas import tpu as pltpu
from jax.experimental.pallas import tpu_sc as plsc

NC = 2
NS = 16
NW = NC * NS

F32 = jnp.float32


def _relu_proj_body(x_ref, w_ref, b_ref, o_ref):
    o_ref[...] = jax.nn.relu(
        jnp.dot(x_ref[...], w_ref[...], preferred_element_type=F32) + b_ref[...])


def _relu_proj(x, w, b, block_rows):
    n, d_in = x.shape
    d_out = w.shape[1]
    grid = n // block_rows
    return pl.pallas_call(
        _relu_proj_body,
        grid=(grid,),
        in_specs=[
            pl.BlockSpec((block_rows, d_in), lambda i: (i, 0)),
            pl.BlockSpec((d_in, d_out), lambda i: (0, 0)),
            pl.BlockSpec((1, d_out), lambda i: (0, 0)),
        ],
        out_specs=pl.BlockSpec((block_rows, d_out), lambda i: (i, 0)),
        out_shape=jax.ShapeDtypeStruct((n, d_out), F32),
    )(x, w, b.reshape(1, d_out))


def _ln(y, g, b):
    m = jnp.mean(y, axis=-1, keepdims=True)
    v = jnp.mean((y - m) ** 2, axis=-1, keepdims=True)
    return (y - m) * lax.rsqrt(v + 1e-5) * g + b


def _edge_layer_body(hs_ref, hd_ref, ea_ref, eh_ref, w1s_ref, w1d_ref, w1e_ref,
                     b1_ref, w2_ref, b2_ref, g_ref, bb_ref, o_ref):
    dt = w1s_ref.dtype
    h = jnp.dot(hs_ref[...].astype(dt), w1s_ref[...], preferred_element_type=F32)
    h = h + jnp.dot(hd_ref[...].astype(dt), w1d_ref[...],
                    preferred_element_type=F32)
    h = h + jnp.dot(ea_ref[...], w1e_ref[...], preferred_element_type=F32)
    h = jax.nn.relu(h + b1_ref[...])
    msg = jnp.dot(h.astype(w2_ref.dtype), w2_ref[...],
                  preferred_element_type=F32) + b2_ref[...]
    y = eh_ref[...] + msg
    o_ref[...] = _ln(y, g_ref[...], bb_ref[...])


def _edge_layer(hs, hd, ea, eh, w1s, w1d, w1e, b1, w2, b2, g, bb, block_rows):
    e, hdim = eh.shape
    hp = hs.shape[1]
    de = ea.shape[1]
    h2 = w2.shape[0]
    grid = e // block_rows
    bf16 = jnp.bfloat16
    return pl.pallas_call(
        _edge_layer_body,
        grid=(grid,),
        in_specs=[
            pl.BlockSpec((block_rows, hp), lambda i: (i, 0)),
            pl.BlockSpec((block_rows, hp), lambda i: (i, 0)),
            pl.BlockSpec((block_rows, de), lambda i: (i, 0)),
            pl.BlockSpec((block_rows, hdim), lambda i: (i, 0)),
            pl.BlockSpec((hp, h2), lambda i: (0, 0)),
            pl.BlockSpec((hp, h2), lambda i: (0, 0)),
            pl.BlockSpec((de, h2), lambda i: (0, 0)),
            pl.BlockSpec((1, h2), lambda i: (0, 0)),
            pl.BlockSpec((h2, hdim), lambda i: (0, 0)),
            pl.BlockSpec((1, hdim), lambda i: (0, 0)),
            pl.BlockSpec((1, hdim), lambda i: (0, 0)),
            pl.BlockSpec((1, hdim), lambda i: (0, 0)),
        ],
        out_specs=pl.BlockSpec((block_rows, hdim), lambda i: (i, 0)),
        out_shape=jax.ShapeDtypeStruct((e, hdim), F32),
    )(hs, hd, ea, eh,
      w1s.astype(bf16), w1d.astype(bf16),
      w1e, b1.reshape(1, h2), w2,
      b2.reshape(1, hdim), g.reshape(1, hdim), bb.reshape(1, hdim))


def _node_layer_body(nh_ref, agg_ref, aggb_ref, w1n_ref, w1a_ref, b1_ref,
                     w2_ref, b2_ref, g_ref, bb_ref, o_ref):
    agg = agg_ref[0] + agg_ref[1] + aggb_ref[0] + aggb_ref[1]
    nh = nh_ref[...]
    h = jnp.dot(nh, w1n_ref[...], preferred_element_type=F32)
    h = h + jnp.dot(agg, w1a_ref[...], preferred_element_type=F32)
    h = jax.nn.relu(h + b1_ref[...])
    upd = jnp.dot(h, w2_ref[...], preferred_element_type=F32) + b2_ref[...]
    o_ref[...] = _ln(nh + upd, g_ref[...], bb_ref[...])


def _node_layer(nh, aggp, aggp2, w1n, w1a, b1, w2, b2, g, bb, block_rows):
    n, hdim = nh.shape
    h2 = w2.shape[0]
    grid = n // block_rows
    return pl.pallas_call(
        _node_layer_body,
        grid=(grid,),
        in_specs=[
            pl.BlockSpec((block_rows, hdim), lambda i: (i, 0)),
            pl.BlockSpec((2, block_rows, hdim), lambda i: (0, i, 0)),
            pl.BlockSpec((2, block_rows, hdim), lambda i: (0, i, 0)),
            pl.BlockSpec((hdim, h2), lambda i: (0, 0)),
            pl.BlockSpec((hdim, h2), lambda i: (0, 0)),
            pl.BlockSpec((1, h2), lambda i: (0, 0)),
            pl.BlockSpec((h2, hdim), lambda i: (0, 0)),
            pl.BlockSpec((1, hdim), lambda i: (0, 0)),
            pl.BlockSpec((1, hdim), lambda i: (0, 0)),
            pl.BlockSpec((1, hdim), lambda i: (0, 0)),
        ],
        out_specs=pl.BlockSpec((block_rows, hdim), lambda i: (i, 0)),
        out_shape=jax.ShapeDtypeStruct((n, hdim), F32),
    )(nh, aggp, aggp2, w1n, w1a, b1.reshape(1, h2), w2, b2.reshape(1, hdim),
      g.reshape(1, hdim), bb.reshape(1, hdim))


def _head_body(lat_ref, sidx_ref, emb0_ref, emb1_ref, h0_ref, irr_ref, rea_ref,
               h0w1_ref, h0b1_ref, h0w2_ref, h0b2_ref,
               t1lat_ref, t1h0_ref, tirr_ref, trea_ref, tb1_ref, tw2_ref,
               tb2_ref, s1p_ref, s1h_ref, sb1_ref, sw2_ref, sb2_ref,
               tok_ref, size_ref):
    sf = sidx_ref[...]
    side = sf * emb1_ref[...] + (1.0 - sf) * emb0_ref[...]
    lath = lat_ref[...] + side
    irr = irr_ref[...]
    rea = rea_ref[...]

    h0 = h0_ref[...]
    h0h = jax.nn.relu(h0 * h0w1_ref[...] + h0b1_ref[...])
    h0_emb = jnp.dot(h0h, h0w2_ref[...], preferred_element_type=F32) + h0b2_ref[...]

    base = jnp.dot(lath, t1lat_ref[...], preferred_element_type=F32)
    h0c = jnp.dot(h0_emb, t1h0_ref[...], preferred_element_type=F32)
    w_irr = tirr_ref[...]
    w_rea = trea_ref[...]
    tw2 = tw2_ref[...]
    nb = irr.shape[0]
    for b in range(nb):
        hid = jax.nn.relu(base + h0c[b] + irr[b][:, None] * w_irr
                          + rea[b][:, None] * w_rea + tb1_ref[...])
        tok_ref[b, :] = jnp.sum(hid * tw2, axis=1) + tb2_ref[0, 0]

    avail = rea * (1.0 - irr)
    denom = jnp.maximum(jnp.sum(avail, axis=1, keepdims=True), 1.0)
    pooled = jnp.dot(avail, lath, preferred_element_type=F32) / denom
    sh = jax.nn.relu(jnp.dot(pooled, s1p_ref[...], preferred_element_type=F32)
                     + jnp.dot(h0_emb, s1h_ref[...], preferred_element_type=F32)
                     + sb1_ref[...])
    size_ref[...] = jnp.dot(sh, sw2_ref[...], preferred_element_type=F32) + sb2_ref[...]


def _head(lat, side_idx, h0, irr_f, rea_f, p):
    l, hdim = lat.shape
    nb = h0.shape[0]
    tok_w1 = p['tok_W1']
    emb = p['side_emb']
    return pl.pallas_call(
        _head_body,
        out_shape=(jax.ShapeDtypeStruct((nb, l), F32),
                   jax.ShapeDtypeStruct((nb, 3), F32)),
    )(lat, side_idx.astype(F32).reshape(l, 1), emb[0].reshape(1, hdim),
      emb[1].reshape(1, hdim), h0.reshape(nb, 1), irr_f, rea_f,
      p['h0_W1'].reshape(1, hdim), p['h0_b1'].reshape(1, hdim), p['h0_W2'],
      p['h0_b2'].reshape(1, hdim),
      tok_w1[:hdim], tok_w1[hdim:2 * hdim], tok_w1[2 * hdim].reshape(1, hdim),
      tok_w1[2 * hdim + 1].reshape(1, hdim), p['tok_b1'].reshape(1, hdim),
      p['tok_W2'].reshape(1, hdim), p['tok_b2'].reshape(1, 1),
      p['size_W1'][:hdim], p['size_W1'][hdim:], p['size_b1'].reshape(1, hdim),
      p['size_W2'], p['size_b2'].reshape(1, 3))


def _gather2(node_h, src, dst, chunk=80, nbuf=5):
    e = src.shape[0]
    hdim = node_h.shape[1]
    dt = node_h.dtype
    per_w = e // NW
    n_chunks = per_w // chunk
    n_groups = n_chunks // nbuf
    assert n_chunks % nbuf == 0
    mesh = plsc.VectorSubcoreMesh(core_axis_name="c", subcore_axis_name="s")

    assert n_groups >= 2

    @functools.partial(
        pl.kernel, mesh=mesh,
        out_type=(jax.ShapeDtypeStruct((e, hdim), dt),
                  jax.ShapeDtypeStruct((e, hdim), dt)),
        scratch_types=[
            pltpu.VMEM((per_w,), jnp.int32),
            pltpu.VMEM((per_w,), jnp.int32),
            pltpu.VMEM((2, nbuf, chunk, hdim), dt),
            pltpu.VMEM((2, nbuf, chunk, hdim), dt),
            pltpu.SemaphoreType.DMA,
            pltpu.SemaphoreType.DMA,
            pltpu.SemaphoreType.DMA,
            pltpu.SemaphoreType.DMA,
        ],
    )
    def k(node_hbm, src_hbm, dst_hbm, hs_hbm, hd_hbm,
          sidx, didx, srows, drows, gsem0, gsem1, wsem0, wsem1):
        wid = lax.axis_index("s") * NC + lax.axis_index("c")
        base = wid * per_w
        pltpu.sync_copy(src_hbm.at[pl.ds(base, per_w)], sidx)
        pltpu.sync_copy(dst_hbm.at[pl.ds(base, per_w)], didx)
        gsems = (gsem0, gsem1)
        wsems = (wsem0, wsem1)

        def fire_g(g, s):
            for b in range(nbuf):
                lo = (g * nbuf + b) * chunk
                pltpu.async_copy(node_hbm.at[sidx.at[pl.ds(lo, chunk)]],
                                 srows.at[s].at[b], gsems[s])
                pltpu.async_copy(node_hbm.at[didx.at[pl.ds(lo, chunk)]],
                                 drows.at[s].at[b], gsems[s])

        def wait_g(s):
            for b in range(nbuf):
                pltpu.make_async_copy(node_hbm.at[sidx.at[pl.ds(0, chunk)]],
                                      srows.at[s].at[b], gsems[s]).wait()
                pltpu.make_async_copy(node_hbm.at[didx.at[pl.ds(0, chunk)]],
                                      drows.at[s].at[b], gsems[s]).wait()

        def fire_w(g, s):
            for b in range(nbuf):
                off = base + (g * nbuf + b) * chunk
                pltpu.async_copy(srows.at[s].at[b],
                                 hs_hbm.at[pl.ds(off, chunk)], wsems[s])
                pltpu.async_copy(drows.at[s].at[b],
                                 hd_hbm.at[pl.ds(off, chunk)], wsems[s])

        def wait_w(s):
            for b in range(nbuf):
                pltpu.make_async_copy(srows.at[s].at[b],
                                      hs_hbm.at[pl.ds(base, chunk)],
                                      wsems[s]).wait()
                pltpu.make_async_copy(drows.at[s].at[b],
                                      hd_hbm.at[pl.ds(base, chunk)],
                                      wsems[s]).wait()

        fire_g(0, 0)
        fire_g(1, 1)

        def pair(kk, carry):
            g0 = 2 * kk
            wait_g(0)
            fire_w(g0, 0)

            @pl.when(g0 + 1 < n_groups)
            def _():
                wait_g(1)
                fire_w(g0 + 1, 1)

            @pl.when(g0 + 2 < n_groups)
            def _():
                wait_w(0)
                fire_g(g0 + 2, 0)

            @pl.when(g0 + 3 < n_groups)
            def _():
                wait_w(1)
                fire_g(g0 + 3, 1)

            return carry

        lax.fori_loop(0, (n_groups + 1) // 2, pair, 0)
        wait_w(0)
        wait_w(1)

    return k(node_h, src, dst)


def _scatter_add(edge_h, dst, n_pad, zeros_hbm, chunk=40, nbuf=5):
    e, hdim = edge_h.shape
    per_w = e // NW
    n_chunks = per_w // chunk
    rows_per_tile = n_pad // NS
    mesh = plsc.VectorSubcoreMesh(core_axis_name="c", subcore_axis_name="s")

    @functools.partial(
        pl.kernel, mesh=mesh,
        out_type=jax.ShapeDtypeStruct((NC, n_pad, hdim), F32),
        scratch_types=[
            pltpu.VMEM((nbuf, chunk), jnp.int32),
            pltpu.VMEM((nbuf, chunk, hdim), F32),
            pltpu.VMEM_SHARED((n_pad, hdim), F32),
            pltpu.SemaphoreType.DMA,
        ],
    )
    def k(eh_hbm, dst_hbm, zero_hbm, out_hbm, idx_v, rows_v, acc_sh, sem):
        c = lax.axis_index("c")
        s = lax.axis_index("s")
        pltpu.sync_copy(zero_hbm.at[pl.ds(s * rows_per_tile, rows_per_tile)],
                        acc_sh.at[pl.ds(s * rows_per_tile, rows_per_tile)])
        plsc.subcore_barrier()

        base = (c * NS + s) * per_w

        def fire(j, b):
            off = base + j * chunk
            pltpu.async_copy(dst_hbm.at[pl.ds(off, chunk)], idx_v.at[b], sem)
            pltpu.async_copy(eh_hbm.at[pl.ds(off, chunk)], rows_v.at[b], sem)

        def wait(b):
            pltpu.make_async_copy(dst_hbm.at[pl.ds(base, chunk)],
                                  idx_v.at[b], sem).wait()
            pltpu.make_async_copy(eh_hbm.at[pl.ds(base, chunk)],
                                  rows_v.at[b], sem).wait()

        for b in range(nbuf):
            fire(b, b)

        def group(g, carry):
            for b in range(nbuf):
                wait(b)
                pltpu.sync_copy(rows_v.at[b], acc_sh.at[idx_v.at[b]], add=True)

                @pl.when(g + 1 < n_chunks // nbuf)
                def _():
                    fire((g + 1) * nbuf + b, b)
            return carry

        lax.fori_loop(0, n_chunks // nbuf, group, 0)
        plsc.subcore_barrier()
        pltpu.sync_copy(acc_sh.at[pl.ds(s * rows_per_tile, rows_per_tile)],
                        out_hbm.at[c, pl.ds(s * rows_per_tile, rows_per_tile)])

    return k(edge_h, dst, zeros_hbm)


def _gather1(node_h, idx):
    l = idx.shape[0]
    hdim = node_h.shape[1]
    per_w = l // NW
    mesh = plsc.VectorSubcoreMesh(core_axis_name="c", subcore_axis_name="s")

    @functools.partial(
        pl.kernel, mesh=mesh,
        out_type=jax.ShapeDtypeStruct((l, hdim), F32),
        scratch_types=[
            pltpu.VMEM((per_w,), jnp.int32),
            pltpu.VMEM((per_w, hdim), F32),
            pltpu.SemaphoreType.DMA,
        ],
    )
    def k(node_hbm, idx_hbm, out_hbm, idx_v, rows_v, sem):
        wid = lax.axis_index("s") * NC + lax.axis_index("c")
        base = wid * per_w
        pltpu.sync_copy(idx_hbm.at[pl.ds(base, per_w)], idx_v)
        pltpu.async_copy(node_hbm.at[idx_v], rows_v, sem).wait()
        pltpu.sync_copy(rows_v, out_hbm.at[pl.ds(base, per_w)])

    return k(node_h, idx)


def kernel(node_x, edge_index, edge_attr, lateral_to_node_idx, side_idx, H0,
           irrigated, reachable, params):
    p = params
    n, _ = node_x.shape
    e = edge_attr.shape[0]
    hdim = p['node_proj_W'].shape[1]
    nl = p['eW1'].shape[0]

    node_h = _relu_proj(node_x, p['node_proj_W'], p['node_proj_b'], 1000)

    eh_ = e // 2
    ea_halves = (edge_attr[:eh_], edge_attr[eh_:])
    src_halves = (edge_index[0, :eh_], edge_index[0, eh_:])
    dst_halves = (edge_index[1, :eh_], edge_index[1, eh_:])
    edge_h = [
        _relu_proj(a, p['edge_proj_W'], p['edge_proj_b'], 4000)
        for a in ea_halves
    ]
    n_pad = ((n + 8 * NS - 1) // (8 * NS)) * (8 * NS)
    zeros_hbm = jnp.zeros((n_pad, hdim), F32)

    bf16 = jnp.bfloat16
    for i in range(nl):
        ew1 = p['eW1'][i]
        aggs = []
        for h in range(2):
            hs, hd = _gather2(node_h, src_halves[h], dst_halves[h], chunk=40)
            edge_h[h] = _edge_layer(
                hs, hd, ea_halves[h], edge_h[h],
                ew1[:hdim], ew1[hdim:2 * hdim],
                ew1[2 * hdim:],
                p['eb1'][i], p['eW2'][i].astype(bf16), p['eb2'][i],
                p['ln_eg'][i], p['ln_eb'][i], 4000)
            aggs.append(_scatter_add(edge_h[h], dst_halves[h], n_pad,
                                     zeros_hbm))
        nw1 = p['nW1'][i]
        node_h = _node_layer(
            node_h, aggs[0], aggs[1], nw1[:hdim], nw1[hdim:],
            p['nb1'][i], p['nW2'][i], p['nb2'][i],
            p['ln_ng'][i], p['ln_nb'][i], 1000)

    lat = _gather1(node_h, lateral_to_node_idx)
    token_logits, size_logits = _head(
        lat, side_idx, H0, irrigated.astype(F32), reachable.astype(F32), p)
    return (token_logits, size_logits)
